# baseline (speedup 1.0000x reference)
import functools

import jax
import jax.numpy as jnp
from jax import lax
from jax.experimental import pallas as pl
from jax.experimental.pallas import tpu as pltpu
from jax.experimental.pallas import tpu_sc as plsc

_K = 64
_B = 32
_N = 10000
_D = 128
_NPAD = 10240
_CHUNK = 128
_NCHUNK = _NPAD // _CHUNK
_MININT = -(2 ** 31)


def _mesh():
    return plsc.VectorSubcoreMesh(core_axis_name="c", subcore_axis_name="s")


def _wid(nc):
    return lax.axis_index("s") * nc + lax.axis_index("c")


def _s32(v):
    b = lax.bitcast_convert_type(v, jnp.int32)
    return jnp.where(b < 0, b ^ jnp.int32(0x7FFFFFFF), b)


@functools.lru_cache(maxsize=1)
def _make_keys_extract():
    info = plsc.get_sparse_core_info()
    nc = info.num_cores

    @functools.partial(
        pl.kernel,
        mesh=_mesh(),
        out_type=jax.ShapeDtypeStruct((_B, _NPAD), jnp.float32),
        scratch_types=[
            pltpu.VMEM((_NPAD,), jnp.int32),
            pltpu.VMEM((_NPAD,), jnp.float32),
            pltpu.SemaphoreType.DMA,
        ],
    )
    def keys_extract(xflat_hbm, idx_hbm, out_hbm, idx_v, keys_v, sem):
        b = _wid(nc)
        pltpu.sync_copy(idx_hbm.at[b], idx_v)

        def fire(c, carry):
            off = c * _CHUNK
            pltpu.async_copy(
                xflat_hbm.at[idx_v.at[pl.ds(off, _CHUNK)]],
                keys_v.at[pl.ds(off, _CHUNK)], sem)
            return carry

        lax.fori_loop(0, _NCHUNK, fire, 0)

        def drain(c, carry):
            off = c * _CHUNK
            pltpu.make_async_copy(
                xflat_hbm.at[idx_v.at[pl.ds(off, _CHUNK)]],
                keys_v.at[pl.ds(off, _CHUNK)], sem).wait()
            return carry

        lax.fori_loop(0, _NCHUNK, drain, 0)

        def tail(t, carry):
            keys_v[pl.ds(_N + t * 16, 16)] = jnp.full(
                (16,), -jnp.inf, jnp.float32)
            return carry

        lax.fori_loop(0, (_NPAD - _N) // 16, tail, 0)
        pltpu.sync_copy(keys_v, out_hbm.at[b])

    return keys_extract


def _thresh_body(keys_ref, out_ref, hint_ref):
    s = _s32(keys_ref[...])
    ones_col = jnp.full((_NPAD, 1), 1.0, jnp.float32)

    def step(j, t_u):
        bit = lax.shift_left(jnp.int32(1), 31 - j)
        try_u = t_u | bit
        ge = (s >= (try_u ^ jnp.int32(_MININT))).astype(jnp.float32)
        cnt = lax.dot_general(ge, ones_col, (((1,), (0,)), ((), ())),
                              preferred_element_type=jnp.float32)
        return jnp.where(cnt >= jnp.float32(_K), try_u, t_u)

    t_u = lax.fori_loop(0, 32, step, jnp.zeros((_B, 1), jnp.int32))
    t_s = t_u ^ jnp.int32(_MININT)
    m_ge = s >= t_s
    cnt_gt = jnp.sum((s > t_s).astype(jnp.int32), axis=1, keepdims=True)
    lane = lax.broadcasted_iota(jnp.int32, (_B, 128), 1)
    out = jnp.where(lane == 0, t_s, jnp.int32(0))
    out = jnp.where(lane == 1, _K - cnt_gt, out)
    out_ref[...] = out
    w = m_ge.astype(jnp.int32)
    for d in (1, 2, 4, 8):
        w = w + jnp.concatenate(
            [jnp.zeros((_B, d), jnp.int32), w[:, :-d]], axis=1)
    hint_ref[...] = w


def _threshold(keys):
    return pl.pallas_call(
        _thresh_body,
        out_shape=[
            jax.ShapeDtypeStruct((_B, 128), jnp.int32),
            jax.ShapeDtypeStruct((_B, _NPAD), jnp.int32),
        ],
    )(keys)


@functools.lru_cache(maxsize=1)
def _make_select_gather():
    info = plsc.get_sparse_core_info()
    nc = info.num_cores

    @functools.partial(
        pl.kernel,
        mesh=_mesh(),
        out_type=jax.ShapeDtypeStruct((_B, _K, _D), jnp.float32),
        scratch_types=[
            pltpu.VMEM((_NPAD,), jnp.float32),
            pltpu.VMEM((_NPAD,), jnp.int32),
            pltpu.VMEM((128,), jnp.int32),
            pltpu.VMEM((_K + 96,), jnp.int32),
            pltpu.VMEM((_K + 96,), jnp.int32),
            pltpu.VMEM((_K + 96,), jnp.int32),
            pltpu.VMEM((_K,), jnp.int32),
            pltpu.VMEM((_K, _D), jnp.float32),
            pltpu.SMEM((_K,), jnp.int32),
            pltpu.SMEM((4,), jnp.int32),
            pltpu.SemaphoreType.DMA,
        ],
    )
    def select_gather(keys_hbm, thr_hbm, hint_hbm, table_hbm,
                      out_hbm, keys_v, hint_v, thr_v, gts_v, gti_v,
                      eqi_v, ord_v, rows_v, ord_sm, cnt_sm, sem):
        b = _wid(nc)
        iota = lax.broadcasted_iota(jnp.int32, (16,), 0)
        cp_k = pltpu.async_copy(keys_hbm.at[b], keys_v, sem)
        cp_h = pltpu.async_copy(hint_hbm.at[b], hint_v, sem)
        cp_t = pltpu.async_copy(thr_hbm.at[b], thr_v, sem)
        cp_k.wait()
        cp_h.wait()
        cp_t.wait()
        tv = thr_v[pl.ds(0, 16)]
        t_s = tv[0]
        need_eq = tv[1]
        cnt_gt = _K - need_eq

        trash = _K + 80
        ones16 = jnp.full((16,), 1, jnp.int32)
        cnt_sm[0] = 0
        cnt_sm[1] = 0

        def scan(j, carry):
            hv = hint_v[pl.ds(j * 16, 16)]

            @pl.when(hv[15] > 0)
            def _():
                s = _s32(keys_v[pl.ds(j * 16, 16)])
                p = cnt_sm[0]
                q = cnt_sm[1]
                for l in range(16):
                    sl = s[l]
                    il16 = ones16 * (j * 16 + l)
                    is_gt = sl > t_s
                    is_eq = sl == t_s
                    dst = jnp.where(is_gt, p, trash)
                    gts_v[pl.ds(dst, 16)] = ones16 * sl
                    gti_v[pl.ds(dst, 16)] = il16
                    edst = jnp.where(is_eq & (q < need_eq), q, trash)
                    eqi_v[pl.ds(edst, 16)] = il16
                    p = p + jnp.where(is_gt, 1, 0)
                    q = q + jnp.where(is_eq, 1, 0)
                cnt_sm[0] = p
                cnt_sm[1] = q

            return carry

        lax.fori_loop(0, _NPAD // 16, scan, 0)

        for v in range(_K // 16):
            gts_v[pl.ds(cnt_gt + v * 16, 16)] = ones16 * t_s
            gti_v[pl.ds(cnt_gt + v * 16, 16)] = eqi_v[pl.ds(v * 16, 16)]

        ranks = [jnp.zeros((16,), jnp.int32) for _ in range(_K // 16)]
        svecs = [gts_v[pl.ds(v * 16, 16)] for v in range(_K // 16)]
        ivecs = [gti_v[pl.ds(v * 16, 16)] for v in range(_K // 16)]

        def rank_step(j, rs):
            sj = gts_v[pl.ds(j, 16)][0]
            ij = gti_v[pl.ds(j, 16)][0]
            out = []
            for v in range(_K // 16):
                beat = (sj > svecs[v]) | ((sj == svecs[v]) & (ij < ivecs[v]))
                out.append(rs[v] + jnp.where(beat, 1, 0))
            return tuple(out)

        ranks = lax.fori_loop(0, _K, rank_step, tuple(ranks))

        for v in range(_K // 16):
            for l in range(16):
                ord_sm[ranks[v][l]] = ivecs[v][l]
        for v in range(_K // 16):
            vec = jnp.zeros((16,), jnp.int32)
            for l in range(16):
                vec = jnp.where(iota == l, ord_sm[v * 16 + l], vec)
            ord_v[pl.ds(v * 16, 16)] = b * _N + vec

        pltpu.async_copy(table_hbm.at[ord_v], rows_v, sem).wait()
        pltpu.sync_copy(rows_v, out_hbm.at[b])

    return select_gather


def kernel(x):
    xflat = x.reshape(_B * _N * _D)
    node = jnp.minimum(jax.lax.iota(jnp.int32, _NPAD), _N - 1)
    gidx = ((jax.lax.iota(jnp.int32, _B)[:, None] * _N + node[None, :])
            * _D + (_D - 1))
    keys = _make_keys_extract()(xflat, gidx)
    thr, hint = _threshold(keys)
    table = x.reshape(_B * _N, _D)
    out = _make_select_gather()(keys, thr, hint, table)
    return out.reshape(_B, _K * _D)


# speedup vs baseline: 1.1383x; 1.1383x over previous
import functools

import jax
import jax.numpy as jnp
from jax import lax
from jax.experimental import pallas as pl
from jax.experimental.pallas import tpu as pltpu
from jax.experimental.pallas import tpu_sc as plsc

_K = 64
_B = 32
_N = 10000
_D = 128
_NPAD = 10240
_CHUNK = 128
_NCHUNK = _NPAD // _CHUNK
_MININT = -(2 ** 31)


def _mesh():
    return plsc.VectorSubcoreMesh(core_axis_name="c", subcore_axis_name="s")


def _wid(nc):
    return lax.axis_index("s") * nc + lax.axis_index("c")


def _s32(v):
    b = lax.bitcast_convert_type(v, jnp.int32)
    return jnp.where(b < 0, b ^ jnp.int32(0x7FFFFFFF), b)


@functools.lru_cache(maxsize=1)
def _make_keys_extract():
    info = plsc.get_sparse_core_info()
    nc = info.num_cores

    @functools.partial(
        pl.kernel,
        mesh=_mesh(),
        out_type=jax.ShapeDtypeStruct((_B, _NPAD), jnp.float32),
        scratch_types=[
            pltpu.VMEM((_NPAD,), jnp.int32),
            pltpu.VMEM((_NPAD,), jnp.float32),
            pltpu.SemaphoreType.DMA,
        ],
    )
    def keys_extract(xflat_hbm, idx_hbm, out_hbm, idx_v, keys_v, sem):
        b = _wid(nc)
        pltpu.sync_copy(idx_hbm.at[b], idx_v)

        def fire(c, carry):
            off = c * _CHUNK
            pltpu.async_copy(
                xflat_hbm.at[idx_v.at[pl.ds(off, _CHUNK)]],
                keys_v.at[pl.ds(off, _CHUNK)], sem)
            return carry

        lax.fori_loop(0, _NCHUNK, fire, 0)

        def drain(c, carry):
            off = c * _CHUNK
            pltpu.make_async_copy(
                xflat_hbm.at[idx_v.at[pl.ds(off, _CHUNK)]],
                keys_v.at[pl.ds(off, _CHUNK)], sem).wait()
            return carry

        lax.fori_loop(0, _NCHUNK, drain, 0)

        def tail(t, carry):
            keys_v[pl.ds(_N + t * 16, 16)] = jnp.full(
                (16,), -jnp.inf, jnp.float32)
            return carry

        lax.fori_loop(0, (_NPAD - _N) // 16, tail, 0)
        pltpu.sync_copy(keys_v, out_hbm.at[b])

    return keys_extract


def _thresh_body(keys_ref, out_ref, hint_ref):
    s = _s32(keys_ref[...])

    def step(j, t_u):
        bit = lax.shift_left(jnp.int32(1), 31 - j)
        try_u = t_u | bit
        ge = (s >= (try_u ^ jnp.int32(_MININT))).astype(jnp.int32)
        q = _NPAD // 4
        cnt = sum(jnp.sum(ge[:, i * q:(i + 1) * q], axis=1, keepdims=True)
                  for i in range(4))
        return jnp.where(cnt >= _K, try_u, t_u)

    t_u = lax.fori_loop(0, 32, step, jnp.zeros((_B, 1), jnp.int32))
    t_s = t_u ^ jnp.int32(_MININT)
    m_ge = s >= t_s
    cnt_gt = jnp.sum((s > t_s).astype(jnp.int32), axis=1, keepdims=True)
    lane = lax.broadcasted_iota(jnp.int32, (_B, 128), 1)
    out = jnp.where(lane == 0, t_s, jnp.int32(0))
    out = jnp.where(lane == 1, _K - cnt_gt, out)
    out_ref[...] = out
    w = m_ge.astype(jnp.int32)
    for d in (1, 2, 4, 8):
        w = w + jnp.concatenate(
            [jnp.zeros((_B, d), jnp.int32), w[:, :-d]], axis=1)
    hint_ref[...] = w


def _threshold(keys):
    return pl.pallas_call(
        _thresh_body,
        out_shape=[
            jax.ShapeDtypeStruct((_B, 128), jnp.int32),
            jax.ShapeDtypeStruct((_B, _NPAD), jnp.int32),
        ],
    )(keys)


@functools.lru_cache(maxsize=1)
def _make_select_gather():
    info = plsc.get_sparse_core_info()
    nc = info.num_cores

    @functools.partial(
        pl.kernel,
        mesh=_mesh(),
        out_type=jax.ShapeDtypeStruct((_B, _K, _D), jnp.float32),
        scratch_types=[
            pltpu.VMEM((_NPAD,), jnp.float32),
            pltpu.VMEM((_NPAD,), jnp.int32),
            pltpu.VMEM((128,), jnp.int32),
            pltpu.VMEM((_K + 96,), jnp.int32),
            pltpu.VMEM((_K + 96,), jnp.int32),
            pltpu.VMEM((_K + 96,), jnp.int32),
            pltpu.VMEM((_K,), jnp.int32),
            pltpu.VMEM((_K, _D), jnp.float32),
            pltpu.SMEM((_K,), jnp.int32),
            pltpu.SMEM((4,), jnp.int32),
            pltpu.SemaphoreType.DMA,
        ],
    )
    def select_gather(keys_hbm, thr_hbm, hint_hbm, table_hbm,
                      out_hbm, keys_v, hint_v, thr_v, gts_v, gti_v,
                      eqi_v, ord_v, rows_v, ord_sm, cnt_sm, sem):
        b = _wid(nc)
        iota = lax.broadcasted_iota(jnp.int32, (16,), 0)
        cp_k = pltpu.async_copy(keys_hbm.at[b], keys_v, sem)
        cp_h = pltpu.async_copy(hint_hbm.at[b], hint_v, sem)
        cp_t = pltpu.async_copy(thr_hbm.at[b], thr_v, sem)
        cp_k.wait()
        cp_h.wait()
        cp_t.wait()
        tv = thr_v[pl.ds(0, 16)]
        t_s = tv[0]
        need_eq = tv[1]
        cnt_gt = _K - need_eq

        trash = _K + 80
        ones16 = jnp.full((16,), 1, jnp.int32)
        cnt_sm[0] = 0
        cnt_sm[1] = 0

        def scan(j, carry):
            hv = hint_v[pl.ds(j * 16, 16)]

            @pl.when(hv[15] > 0)
            def _():
                s = _s32(keys_v[pl.ds(j * 16, 16)])
                p = cnt_sm[0]
                q = cnt_sm[1]
                for l in range(16):
                    sl = s[l]
                    il16 = ones16 * (j * 16 + l)
                    is_gt = sl > t_s
                    is_eq = sl == t_s
                    dst = jnp.where(is_gt, p, trash)
                    gts_v[pl.ds(dst, 16)] = ones16 * sl
                    gti_v[pl.ds(dst, 16)] = il16
                    edst = jnp.where(is_eq & (q < need_eq), q, trash)
                    eqi_v[pl.ds(edst, 16)] = il16
                    p = p + jnp.where(is_gt, 1, 0)
                    q = q + jnp.where(is_eq, 1, 0)
                cnt_sm[0] = p
                cnt_sm[1] = q

            return carry

        lax.fori_loop(0, _NPAD // 16, scan, 0)

        for v in range(_K // 16):
            gts_v[pl.ds(cnt_gt + v * 16, 16)] = ones16 * t_s
            gti_v[pl.ds(cnt_gt + v * 16, 16)] = eqi_v[pl.ds(v * 16, 16)]

        ranks = [jnp.zeros((16,), jnp.int32) for _ in range(_K // 16)]
        svecs = [gts_v[pl.ds(v * 16, 16)] for v in range(_K // 16)]
        ivecs = [gti_v[pl.ds(v * 16, 16)] for v in range(_K // 16)]

        def rank_step(j, rs):
            sj = gts_v[pl.ds(j, 16)][0]
            ij = gti_v[pl.ds(j, 16)][0]
            out = []
            for v in range(_K // 16):
                beat = (sj > svecs[v]) | ((sj == svecs[v]) & (ij < ivecs[v]))
                out.append(rs[v] + jnp.where(beat, 1, 0))
            return tuple(out)

        ranks = lax.fori_loop(0, _K, rank_step, tuple(ranks))

        for v in range(_K // 16):
            for l in range(16):
                ord_sm[ranks[v][l]] = ivecs[v][l]
        for v in range(_K // 16):
            vec = jnp.zeros((16,), jnp.int32)
            for l in range(16):
                vec = jnp.where(iota == l, ord_sm[v * 16 + l], vec)
            ord_v[pl.ds(v * 16, 16)] = b * _N + vec

        pltpu.async_copy(table_hbm.at[ord_v], rows_v, sem).wait()
        pltpu.sync_copy(rows_v, out_hbm.at[b])

    return select_gather


def kernel(x):
    xflat = x.reshape(_B * _N * _D)
    node = jnp.minimum(jax.lax.iota(jnp.int32, _NPAD), _N - 1)
    gidx = ((jax.lax.iota(jnp.int32, _B)[:, None] * _N + node[None, :])
            * _D + (_D - 1))
    keys = _make_keys_extract()(xflat, gidx)
    thr, hint = _threshold(keys)
    table = x.reshape(_B * _N, _D)
    out = _make_select_gather()(keys, thr, hint, table)
    return out.reshape(_B, _K * _D)


# speedup vs baseline: 1.1567x; 1.0162x over previous
import functools

import jax
import jax.numpy as jnp
from jax import lax
from jax.experimental import pallas as pl
from jax.experimental.pallas import tpu as pltpu
from jax.experimental.pallas import tpu_sc as plsc

_K = 64
_B = 32
_N = 10000
_D = 128
_NPAD = 10240
_CHUNK = 128
_NCHUNK = _NPAD // _CHUNK
_MININT = -(2 ** 31)


def _mesh():
    return plsc.VectorSubcoreMesh(core_axis_name="c", subcore_axis_name="s")


def _wid(nc):
    return lax.axis_index("s") * nc + lax.axis_index("c")


def _s32(v):
    b = lax.bitcast_convert_type(v, jnp.int32)
    return jnp.where(b < 0, b ^ jnp.int32(0x7FFFFFFF), b)


@functools.lru_cache(maxsize=1)
def _make_keys_extract():
    info = plsc.get_sparse_core_info()
    nc = info.num_cores

    @functools.partial(
        pl.kernel,
        mesh=_mesh(),
        out_type=jax.ShapeDtypeStruct((_B, _NPAD), jnp.float32),
        scratch_types=[
            pltpu.VMEM((_NPAD,), jnp.int32),
            pltpu.VMEM((_NPAD,), jnp.float32),
            pltpu.SemaphoreType.DMA,
        ],
    )
    def keys_extract(xflat_hbm, idx_hbm, out_hbm, idx_v, keys_v, sem):
        b = _wid(nc)
        pltpu.sync_copy(idx_hbm.at[b], idx_v)

        def fire(c, carry):
            off = c * _CHUNK
            pltpu.async_copy(
                xflat_hbm.at[idx_v.at[pl.ds(off, _CHUNK)]],
                keys_v.at[pl.ds(off, _CHUNK)], sem)
            return carry

        lax.fori_loop(0, _NCHUNK, fire, 0)

        def drain(c, carry):
            off = c * _CHUNK
            pltpu.make_async_copy(
                xflat_hbm.at[idx_v.at[pl.ds(off, _CHUNK)]],
                keys_v.at[pl.ds(off, _CHUNK)], sem).wait()
            return carry

        lax.fori_loop(0, _NCHUNK, drain, 0)

        def tail(t, carry):
            keys_v[pl.ds(_N + t * 16, 16)] = jnp.full(
                (16,), -jnp.inf, jnp.float32)
            return carry

        lax.fori_loop(0, (_NPAD - _N) // 16, tail, 0)
        pltpu.sync_copy(keys_v, out_hbm.at[b])

    return keys_extract


def _thresh_body(keys_ref, out_ref, hint_ref):
    s = _s32(keys_ref[...])

    def step(j, t_u):
        bit = lax.shift_left(jnp.int32(1), 31 - j)
        try_u = t_u | bit
        ge = (s >= (try_u ^ jnp.int32(_MININT))).astype(jnp.int32)
        q = _NPAD // 4
        cnt = sum(jnp.sum(ge[:, i * q:(i + 1) * q], axis=1, keepdims=True)
                  for i in range(4))
        return jnp.where(cnt >= _K, try_u, t_u)

    t_u = lax.fori_loop(0, 32, step, jnp.zeros((_B, 1), jnp.int32))
    t_s = t_u ^ jnp.int32(_MININT)
    m_gt = s > t_s
    m_eq = s == t_s
    cnt_gt = jnp.sum(m_gt.astype(jnp.int32), axis=1, keepdims=True)
    lane = lax.broadcasted_iota(jnp.int32, (_B, 128), 1)
    out = jnp.where(lane == 0, t_s, jnp.int32(0))
    out = jnp.where(lane == 1, _K - cnt_gt, out)
    out_ref[...] = out
    lanemod = lax.broadcasted_iota(jnp.int32, (_B, _NPAD), 1) & 15
    c = jnp.where(m_gt, jnp.int32(1 << 16), jnp.int32(0)) + \
        jnp.where(m_eq, jnp.int32(1), jnp.int32(0))
    for d in (1, 2, 4, 8):
        shifted = jnp.concatenate(
            [jnp.zeros((_B, d), jnp.int32), c[:, :-d]], axis=1)
        c = c + jnp.where(lanemod >= d, shifted, jnp.int32(0))
    hint_ref[...] = c


def _threshold(keys):
    return pl.pallas_call(
        _thresh_body,
        out_shape=[
            jax.ShapeDtypeStruct((_B, 128), jnp.int32),
            jax.ShapeDtypeStruct((_B, _NPAD), jnp.int32),
        ],
    )(keys)


@functools.lru_cache(maxsize=1)
def _make_select_gather():
    info = plsc.get_sparse_core_info()
    nc = info.num_cores

    @functools.partial(
        pl.kernel,
        mesh=_mesh(),
        out_type=jax.ShapeDtypeStruct((_B, _K, _D), jnp.float32),
        scratch_types=[
            pltpu.VMEM((_NPAD,), jnp.float32),
            pltpu.VMEM((_NPAD,), jnp.int32),
            pltpu.VMEM((128,), jnp.int32),
            pltpu.VMEM((_K + 96,), jnp.int32),
            pltpu.VMEM((_K + 96,), jnp.int32),
            pltpu.VMEM((_K + 96,), jnp.int32),
            pltpu.VMEM((_K,), jnp.int32),
            pltpu.VMEM((_K, _D), jnp.float32),
            pltpu.SMEM((_K,), jnp.int32),
            pltpu.SMEM((4,), jnp.int32),
            pltpu.SemaphoreType.DMA,
        ],
    )
    def select_gather(keys_hbm, thr_hbm, hint_hbm, table_hbm,
                      out_hbm, keys_v, hint_v, thr_v, gts_v, gti_v,
                      eqi_v, ord_v, rows_v, ord_sm, cnt_sm, sem):
        b = _wid(nc)
        iota = lax.broadcasted_iota(jnp.int32, (16,), 0)
        cp_k = pltpu.async_copy(keys_hbm.at[b], keys_v, sem)
        cp_h = pltpu.async_copy(hint_hbm.at[b], hint_v, sem)
        cp_t = pltpu.async_copy(thr_hbm.at[b], thr_v, sem)
        cp_k.wait()
        cp_h.wait()
        cp_t.wait()
        tv = thr_v[pl.ds(0, 16)]
        t_s = tv[0]
        need_eq = tv[1]
        cnt_gt = _K - need_eq

        trash = _K + 80
        ones16 = jnp.full((16,), 1, jnp.int32)
        cnt_sm[0] = 0
        cnt_sm[1] = 0

        def scan(j, carry):
            hv = hint_v[pl.ds(j * 16, 16)]

            @pl.when(hv[15] > 0)
            def _():
                s = _s32(keys_v[pl.ds(j * 16, 16)])
                p = cnt_sm[0]
                q = cnt_sm[1]
                pgl = lax.shift_right_logical(hv, 16)
                pel = hv & jnp.int32(0xFFFF)
                m_gt = s > t_s
                dpos = p + pgl - 1
                epos = q + pel - 1
                dstv = jnp.where(m_gt, dpos, trash)
                edstv = jnp.where((s == t_s) & (epos < need_eq),
                                  epos, trash)
                for l in range(16):
                    il16 = ones16 * (j * 16 + l)
                    gts_v[pl.ds(dstv[l], 16)] = ones16 * s[l]
                    gti_v[pl.ds(dstv[l], 16)] = il16
                    eqi_v[pl.ds(edstv[l], 16)] = il16
                cnt_sm[0] = p + pgl[15]
                cnt_sm[1] = q + pel[15]

            return carry

        lax.fori_loop(0, _NPAD // 16, scan, 0)

        for v in range(_K // 16):
            gts_v[pl.ds(cnt_gt + v * 16, 16)] = ones16 * t_s
            gti_v[pl.ds(cnt_gt + v * 16, 16)] = eqi_v[pl.ds(v * 16, 16)]

        ranks = [jnp.zeros((16,), jnp.int32) for _ in range(_K // 16)]
        svecs = [gts_v[pl.ds(v * 16, 16)] for v in range(_K // 16)]
        ivecs = [gti_v[pl.ds(v * 16, 16)] for v in range(_K // 16)]

        def rank_step(j, rs):
            sj = gts_v[pl.ds(j, 16)][0]
            ij = gti_v[pl.ds(j, 16)][0]
            out = []
            for v in range(_K // 16):
                beat = (sj > svecs[v]) | ((sj == svecs[v]) & (ij < ivecs[v]))
                out.append(rs[v] + jnp.where(beat, 1, 0))
            return tuple(out)

        ranks = lax.fori_loop(0, _K, rank_step, tuple(ranks))

        for v in range(_K // 16):
            for l in range(16):
                ord_sm[ranks[v][l]] = ivecs[v][l]
        for v in range(_K // 16):
            vec = jnp.zeros((16,), jnp.int32)
            for l in range(16):
                vec = jnp.where(iota == l, ord_sm[v * 16 + l], vec)
            ord_v[pl.ds(v * 16, 16)] = b * _N + vec

        pltpu.async_copy(table_hbm.at[ord_v], rows_v, sem).wait()
        pltpu.sync_copy(rows_v, out_hbm.at[b])

    return select_gather


def kernel(x):
    xflat = x.reshape(_B * _N * _D)
    node = jnp.minimum(jax.lax.iota(jnp.int32, _NPAD), _N - 1)
    gidx = ((jax.lax.iota(jnp.int32, _B)[:, None] * _N + node[None, :])
            * _D + (_D - 1))
    keys = _make_keys_extract()(xflat, gidx)
    thr, hint = _threshold(keys)
    table = x.reshape(_B * _N, _D)
    out = _make_select_gather()(keys, thr, hint, table)
    return out.reshape(_B, _K * _D)
